# Initial kernel scaffold; baseline (speedup 1.0000x reference)
import jax
import jax.numpy as jnp
from jax.experimental import pallas as pl


def kernel(x, edge_index, edge_attr, batch, edge_params, node1_params, node2_params, global_params, Wp, bp):
    raise NotImplementedError("write your pallas kernel here")


# speedup vs baseline: 2.3781x; 2.3781x over previous
import functools

import jax
import jax.numpy as jnp
from jax import lax
from jax.experimental import pallas as pl
from jax.experimental.pallas import tpu as pltpu
import jax.experimental.pallas.tpu_sc as plsc

N = 50000
E = 1600000
NODE_IN = 9
EDGE_IN = 12
DH = 64
G = 128
OUTDIM = 6
EPS = 1e-5
SLOPE = 0.1

NP = 50176
E2 = E // 2
E4 = E // 4
E8 = E // 8
BE = 6400
NEB = E // BE
BE2 = BE // 2
BE4 = BE // 4
BE8 = BE // 8
NB = 6272
NNB = NP // NB
CH = 2000
CHZ = 1000
CHS = 800
EPW = E // 32
EPW2 = E // 16
ZCH = NP // 16
HZ = ZCH // 2
HZ4 = ZCH // 4

_f32 = jnp.float32


def _mesh():
    return plsc.VectorSubcoreMesh(core_axis_name="c", subcore_axis_name="s")


_SC_PARAMS = pltpu.CompilerParams(use_tc_tiling_on_sc=False)


def _sc_deg(row, col, ones_h_in, zeros_h_in):
    @functools.partial(
        pl.kernel,
        out_type=(jax.ShapeDtypeStruct((2 * NP,), _f32),
                  jax.ShapeDtypeStruct((2 * NP,), _f32)),
        mesh=_mesh(),
        compiler_params=_SC_PARAMS,
        scratch_types=[
            pltpu.VMEM((CH,), jnp.int32),
            pltpu.VMEM((CH,), _f32),
            pltpu.VMEM((ZCH,), _f32),
            pltpu.VMEM_SHARED((NP,), _f32),
            pltpu.VMEM_SHARED((NP,), _f32),
        ],
    )
    def k(row_h, col_h, ones_h, zer_h, outr_h, outc_h,
          idx_v, ones_v, zer_v, acc_r, acc_c):
        cid = lax.axis_index("c")
        sid = lax.axis_index("s")
        wid = sid * 2 + cid
        pltpu.sync_copy(zer_h, zer_v)
        pltpu.sync_copy(zer_v, acc_r.at[pl.ds(sid * ZCH, ZCH)])
        pltpu.sync_copy(zer_v, acc_c.at[pl.ds(sid * ZCH, ZCH)])
        pltpu.sync_copy(ones_h, ones_v)
        plsc.subcore_barrier()

        def step(i, carry):
            base = wid * EPW + i * CH
            pltpu.sync_copy(row_h.at[pl.ds(base, CH)], idx_v)
            pltpu.sync_copy(ones_v, acc_r.at[idx_v], add=True)
            pltpu.sync_copy(col_h.at[pl.ds(base, CH)], idx_v)
            pltpu.sync_copy(ones_v, acc_c.at[idx_v], add=True)
            return carry

        lax.fori_loop(0, EPW // CH, step, 0)
        plsc.subcore_barrier()
        pltpu.sync_copy(acc_r.at[pl.ds(sid * ZCH, ZCH)], zer_v)
        pltpu.sync_copy(zer_v, outr_h.at[pl.ds(cid * NP + sid * ZCH, ZCH)])
        pltpu.sync_copy(acc_c.at[pl.ds(sid * ZCH, ZCH)], zer_v)
        pltpu.sync_copy(zer_v, outc_h.at[pl.ds(cid * NP + sid * ZCH, ZCH)])

    return k(row, col, ones_h_in, zeros_h_in)


def _sc_gath(Ps, Pd, row, col):
    @functools.partial(
        pl.kernel,
        out_type=(jax.ShapeDtypeStruct((E, DH), _f32),
                  jax.ShapeDtypeStruct((E, DH), _f32)),
        mesh=_mesh(),
        compiler_params=_SC_PARAMS,
        scratch_types=[
            pltpu.VMEM((CHZ,), jnp.int32),
            pltpu.VMEM((CHZ, DH), _f32),
            pltpu.SemaphoreType.DMA,
        ],
    )
    def k(ps_h, pd_h, row_h, col_h, zs_h, zd_h, idx_v, rows_v, sem):
        cid = lax.axis_index("c")
        sid = lax.axis_index("s")
        wid = sid * 2 + cid

        def step(i, carry):
            base = wid * EPW + i * CHZ
            pltpu.sync_copy(row_h.at[pl.ds(base, CHZ)], idx_v)
            pltpu.async_copy(ps_h.at[idx_v], rows_v, sem).wait()
            pltpu.sync_copy(rows_v, zs_h.at[pl.ds(base, CHZ)])
            pltpu.sync_copy(col_h.at[pl.ds(base, CHZ)], idx_v)
            pltpu.async_copy(pd_h.at[idx_v], rows_v, sem).wait()
            pltpu.sync_copy(rows_v, zd_h.at[pl.ds(base, CHZ)])
            return carry

        lax.fori_loop(0, EPW // CHZ, step, 0)

    return k(Ps, Pd, row, col)


def _sc_scatter(col, h_lo, h_hi, zeros_h_in):
    @functools.partial(
        pl.kernel,
        out_type=jax.ShapeDtypeStruct((2, NP, 32), _f32),
        mesh=_mesh(),
        compiler_params=_SC_PARAMS,
        scratch_types=[
            pltpu.VMEM((CHS,), jnp.int32),
            pltpu.VMEM((CHS, 32), _f32),
            pltpu.VMEM_SHARED((NP, 32), _f32),
        ],
    )
    def k(col_h, lo_h, hi_h, zer_h, out_h, idx_v, upd_v, acc):
        cid = lax.axis_index("c")
        sid = lax.axis_index("s")
        pltpu.sync_copy(zer_h, upd_v.at[pl.ds(0, HZ4)])
        for kk in range(4):
            pltpu.sync_copy(upd_v.at[pl.ds(0, HZ4)],
                            acc.at[pl.ds(sid * ZCH + kk * HZ4, HZ4)])
        plsc.subcore_barrier()

        def step_from(h_ref):
            def step(i, carry):
                base = sid * EPW2 + i * CHS
                pltpu.sync_copy(col_h.at[pl.ds(base, CHS)], idx_v)
                pltpu.sync_copy(h_ref.at[pl.ds(base, CHS)], upd_v)
                pltpu.sync_copy(upd_v, acc.at[idx_v], add=True)
                return carry
            return step

        @pl.when(cid == 0)
        def _():
            lax.fori_loop(0, EPW2 // CHS, step_from(lo_h), 0)

        @pl.when(cid == 1)
        def _():
            lax.fori_loop(0, EPW2 // CHS, step_from(hi_h), 0)

        plsc.subcore_barrier()
        for kk in range(4):
            pltpu.sync_copy(acc.at[pl.ds(sid * ZCH + kk * HZ4, HZ4)],
                            upd_v.at[pl.ds(0, HZ4)])
            pltpu.sync_copy(upd_v.at[pl.ds(0, HZ4)],
                            out_h.at[cid, pl.ds(sid * ZCH + kk * HZ4, HZ4)])

    return k(col, h_lo, h_hi, zeros_h_in)


def _dot(a, b):
    return jnp.dot(a, b, preferred_element_type=_f32)


def _dgT(w, hT):
    return lax.dot_general(w, hT, (((0,), (0,)), ((), ())),
                           preferred_element_type=_f32)


def _lrelu(z):
    return jnp.where(z > 0, z, SLOPE * z)


def _rows8(*rows):
    w = rows[0].shape[0]
    pad = jnp.zeros((8 - len(rows), w), _f32)
    return jnp.concatenate([r[None] for r in rows] + [pad], axis=0)


def _stats_blk(h):
    return _rows8(jnp.sum(h, axis=0), jnp.sum(h * h, axis=0))


def _acc(ref, blk, i):
    @pl.when(i == 0)
    def _():
        ref[...] = blk

    @pl.when(i > 0)
    def _():
        ref[...] += blk


def _tc_attr_moments(attrT):
    def body(a_ref, st_ref):
        i = pl.program_id(0)
        a = a_ref[...]
        blk = jnp.concatenate(
            [jnp.sum(a, axis=1)[:, None], jnp.sum(a * a, axis=1)[:, None],
             jnp.zeros((EDGE_IN, 14), _f32)], axis=1)
        _acc(st_ref, blk, i)

    return pl.pallas_call(
        body,
        grid=(NEB,),
        in_specs=[pl.BlockSpec((EDGE_IN, BE), lambda i: (0, i))],
        out_specs=pl.BlockSpec((EDGE_IN, 16), lambda i: (0, 0)),
        out_shape=jax.ShapeDtypeStruct((EDGE_IN, 16), _f32),
    )(attrT)


def _tc_node_moments(xT, degs):
    def body(x_ref, d_ref, out_ref):
        xv = x_ref[...]
        deg_r = d_ref[0:1, :] + d_ref[2:3, :]
        deg_c = d_ref[1:2, :] + d_ref[3:4, :]
        out_ref[...] = _rows8(jnp.sum(xv * deg_r, axis=1),
                              jnp.sum(xv * xv * deg_r, axis=1),
                              jnp.sum(xv * deg_c, axis=1),
                              jnp.sum(xv * xv * deg_c, axis=1))

    return pl.pallas_call(
        body,
        out_shape=jax.ShapeDtypeStruct((8, 16), _f32),
    )(xT, degs)


def _tc_proj(xT, Ws, Wd):
    def body(x_ref, ws_ref, wd_ref, ps_ref, pd_ref):
        ps_ref[...] = _dgT(ws_ref[...], x_ref[...])
        pd_ref[...] = _dgT(wd_ref[...], x_ref[...])

    return pl.pallas_call(
        body,
        grid=(NNB,),
        in_specs=[
            pl.BlockSpec((16, NB), lambda i: (0, i)),
            pl.BlockSpec((16, DH), lambda i: (0, 0)),
            pl.BlockSpec((16, DH), lambda i: (0, 0)),
        ],
        out_specs=[
            pl.BlockSpec((DH, NB), lambda i: (0, i)),
            pl.BlockSpec((DH, NB), lambda i: (0, i)),
        ],
        out_shape=[
            jax.ShapeDtypeStruct((DH, NP), _f32),
            jax.ShapeDtypeStruct((DH, NP), _f32),
        ],
    )(xT, Ws, Wd)


def _unpack_T(blk):
    lt = jnp.swapaxes(blk[:, :DH], 0, 1)
    rt = jnp.swapaxes(blk[:, DH:], 0, 1)
    return jnp.concatenate([lt, rt], axis=1)


def _stats_T(h):
    return _rows8(jnp.sum(h, axis=1), jnp.sum(h * h, axis=1))


def _tc_edge1(zs_pk, zd_pk, attrTp, Wa, ccol):
    def body(zs_ref, zd_ref, a_ref, wa_ref, c_ref, h_ref, st_ref):
        i = pl.program_id(0)
        zT = _unpack_T(zs_ref[...] + zd_ref[...])
        h = _lrelu(zT + _dgT(wa_ref[...], a_ref[...]) + c_ref[...])
        h_ref[...] = h
        _acc(st_ref, _stats_T(h), i)

    return pl.pallas_call(
        body,
        grid=(NEB,),
        in_specs=[
            pl.BlockSpec((BE2, 128), lambda i: (i, 0)),
            pl.BlockSpec((BE2, 128), lambda i: (i, 0)),
            pl.BlockSpec((EDGE_IN, BE), lambda i: (0, i)),
            pl.BlockSpec((EDGE_IN, DH), lambda i: (0, 0)),
            pl.BlockSpec((DH, 1), lambda i: (0, 0)),
        ],
        out_specs=[
            pl.BlockSpec((DH, BE), lambda i: (0, i)),
            pl.BlockSpec((8, DH), lambda i: (0, 0)),
        ],
        out_shape=[
            jax.ShapeDtypeStruct((DH, E), _f32),
            jax.ShapeDtypeStruct((8, DH), _f32),
        ],
    )(zs_pk, zd_pk, attrTp, Wa, ccol)


def _tc_edge2(h1T, W, ccol):
    def body(h1_ref, w_ref, c_ref, h_ref, st_ref, g_ref):
        i = pl.program_id(0)
        h = _lrelu(_dgT(w_ref[...], h1_ref[...]) + c_ref[...])
        h_ref[...] = h
        _acc(st_ref, _stats_T(h), i)
        gram = lax.dot_general(h, h, (((1,), (1,)), ((), ())),
                               preferred_element_type=_f32)
        _acc(g_ref, gram, i)

    return pl.pallas_call(
        body,
        grid=(NEB,),
        in_specs=[
            pl.BlockSpec((DH, BE), lambda i: (0, i)),
            pl.BlockSpec((DH, DH), lambda i: (0, 0)),
            pl.BlockSpec((DH, 1), lambda i: (0, 0)),
        ],
        out_specs=[
            pl.BlockSpec((DH, BE), lambda i: (0, i)),
            pl.BlockSpec((8, DH), lambda i: (0, 0)),
            pl.BlockSpec((DH, DH), lambda i: (0, 0)),
        ],
        out_shape=[
            jax.ShapeDtypeStruct((DH, E), _f32),
            jax.ShapeDtypeStruct((8, DH), _f32),
            jax.ShapeDtypeStruct((DH, DH), _f32),
        ],
    )(h1T, W, ccol)


def _tc_node1(zs_pk, h2T, Mn, B, ccol):
    def body(zs_ref, h2_ref, m_ref, b_ref, c_ref, h_ref, st_ref):
        i = pl.program_id(0)
        zsT = _unpack_T(zs_ref[...])
        h = _lrelu(_dgT(m_ref[...], zsT) + _dgT(b_ref[...], h2_ref[...])
                   + c_ref[...])
        h_ref[...] = h
        _acc(st_ref, _stats_T(h), i)

    return pl.pallas_call(
        body,
        grid=(NEB,),
        in_specs=[
            pl.BlockSpec((BE2, 128), lambda i: (i, 0)),
            pl.BlockSpec((DH, BE), lambda i: (0, i)),
            pl.BlockSpec((DH, DH), lambda i: (0, 0)),
            pl.BlockSpec((DH, DH), lambda i: (0, 0)),
            pl.BlockSpec((DH, 1), lambda i: (0, 0)),
        ],
        out_specs=[
            pl.BlockSpec((DH, BE), lambda i: (0, i)),
            pl.BlockSpec((8, DH), lambda i: (0, 0)),
        ],
        out_shape=[
            jax.ShapeDtypeStruct((DH, E), _f32),
            jax.ShapeDtypeStruct((8, DH), _f32),
        ],
    )(zs_pk, h2T, Mn, B, ccol)


def _tc_node2(h1T, W, ccol):
    def body(h1_ref, w_ref, c_ref, lo_ref, hi_ref, st_ref):
        i = pl.program_id(0)
        h = _lrelu(_dgT(w_ref[...], h1_ref[...]) + c_ref[...])
        _acc(st_ref, _stats_T(h), i)
        ht = jnp.swapaxes(h, 0, 1)
        lo = ht[:, :32]
        hi = ht[:, 32:]
        lo_ref[...] = jnp.concatenate(
            [lo[k * BE4:(k + 1) * BE4] for k in range(4)], axis=1)
        hi_ref[...] = jnp.concatenate(
            [hi[k * BE4:(k + 1) * BE4] for k in range(4)], axis=1)

    return pl.pallas_call(
        body,
        grid=(NEB,),
        in_specs=[
            pl.BlockSpec((DH, BE), lambda i: (0, i)),
            pl.BlockSpec((DH, DH), lambda i: (0, 0)),
            pl.BlockSpec((DH, 1), lambda i: (0, 0)),
        ],
        out_specs=[
            pl.BlockSpec((BE4, 128), lambda i: (i, 0)),
            pl.BlockSpec((BE4, 128), lambda i: (i, 0)),
            pl.BlockSpec((8, DH), lambda i: (0, 0)),
        ],
        out_shape=[
            jax.ShapeDtypeStruct((E4, 128), _f32),
            jax.ShapeDtypeStruct((E4, 128), _f32),
            jax.ShapeDtypeStruct((8, DH), _f32),
        ],
    )(h1T, W, ccol)


def _tc_agg(SloT, ShiT, degs, xT, Wlo, Whi, cc):
    def body(lo_ref, hi_ref, d_ref, x_ref, wl_ref, wh_ref, c_ref,
             agg_ref, st_ref, xst_ref):
        i = pl.program_id(0)
        cnt = d_ref[1:2, :] + d_ref[3:4, :]
        sm = _dgT(wl_ref[...], lo_ref[...]) + _dgT(wh_ref[...], hi_ref[...])
        sm = sm + c_ref[...] * cnt
        agg = sm / jnp.maximum(cnt, 1.0)
        agg_ref[...] = agg
        _acc(st_ref, _rows8(jnp.sum(agg, axis=1), jnp.sum(agg * agg, axis=1)), i)
        xv = x_ref[...]
        _acc(xst_ref, _rows8(jnp.sum(xv, axis=1), jnp.sum(xv * xv, axis=1)), i)

    return pl.pallas_call(
        body,
        grid=(NNB,),
        in_specs=[
            pl.BlockSpec((32, NB), lambda i: (0, i)),
            pl.BlockSpec((32, NB), lambda i: (0, i)),
            pl.BlockSpec((4, NB), lambda i: (0, i)),
            pl.BlockSpec((16, NB), lambda i: (0, i)),
            pl.BlockSpec((32, DH), lambda i: (0, 0)),
            pl.BlockSpec((32, DH), lambda i: (0, 0)),
            pl.BlockSpec((DH, 1), lambda i: (0, 0)),
        ],
        out_specs=[
            pl.BlockSpec((DH, NB), lambda i: (0, i)),
            pl.BlockSpec((8, DH), lambda i: (0, 0)),
            pl.BlockSpec((8, 16), lambda i: (0, 0)),
        ],
        out_shape=[
            jax.ShapeDtypeStruct((DH, NP), _f32),
            jax.ShapeDtypeStruct((8, DH), _f32),
            jax.ShapeDtypeStruct((8, 16), _f32),
        ],
    )(SloT, ShiT, degs, xT, Wlo, Whi, cc)


def _tc_nmlp_a(xT, aggT, Ax, Aagg, cc):
    def body(x_ref, agg_ref, ax_ref, aa_ref, c_ref, h_ref, st_ref):
        i = pl.program_id(0)
        h = _lrelu(_dgT(ax_ref[...], x_ref[...]) + _dgT(aa_ref[...], agg_ref[...])
                   + c_ref[...])
        h_ref[...] = h
        pos = lax.broadcasted_iota(jnp.int32, (DH, NB), 1) + i * NB
        hm = jnp.where(pos < N, h, 0.0)
        _acc(st_ref, _rows8(jnp.sum(hm, axis=1), jnp.sum(hm * hm, axis=1)), i)

    return pl.pallas_call(
        body,
        grid=(NNB,),
        in_specs=[
            pl.BlockSpec((16, NB), lambda i: (0, i)),
            pl.BlockSpec((DH, NB), lambda i: (0, i)),
            pl.BlockSpec((16, DH), lambda i: (0, 0)),
            pl.BlockSpec((DH, DH), lambda i: (0, 0)),
            pl.BlockSpec((DH, 1), lambda i: (0, 0)),
        ],
        out_specs=[
            pl.BlockSpec((DH, NB), lambda i: (0, i)),
            pl.BlockSpec((8, DH), lambda i: (0, 0)),
        ],
        out_shape=[
            jax.ShapeDtypeStruct((DH, NP), _f32),
            jax.ShapeDtypeStruct((8, DH), _f32),
        ],
    )(xT, aggT, Ax, Aagg, cc)


def _tc_nmlp_b(hT, W, cc):
    def body(h_ref, w_ref, c_ref, o_ref, st_ref):
        i = pl.program_id(0)
        h = _lrelu(_dgT(w_ref[...], h_ref[...]) + c_ref[...])
        o_ref[...] = h
        pos = lax.broadcasted_iota(jnp.int32, (DH, NB), 1) + i * NB
        hm = jnp.where(pos < N, h, 0.0)
        _acc(st_ref, _rows8(jnp.sum(hm, axis=1), jnp.sum(hm * hm, axis=1)), i)

    return pl.pallas_call(
        body,
        grid=(NNB,),
        in_specs=[
            pl.BlockSpec((DH, NB), lambda i: (0, i)),
            pl.BlockSpec((DH, DH), lambda i: (0, 0)),
            pl.BlockSpec((DH, 1), lambda i: (0, 0)),
        ],
        out_specs=[
            pl.BlockSpec((DH, NB), lambda i: (0, i)),
            pl.BlockSpec((8, DH), lambda i: (0, 0)),
        ],
        out_shape=[
            jax.ShapeDtypeStruct((DH, NP), _f32),
            jax.ShapeDtypeStruct((8, DH), _f32),
        ],
    )(hT, W, cc)


def _tc_pool(h2T, W, cc, batchi):
    def body(h_ref, w_ref, c_ref, b_ref, su_ref, cu_ref):
        i = pl.program_id(0)
        x2 = _dgT(w_ref[...], h_ref[...]) + c_ref[...]
        bb = b_ref[0:1, :]
        gi = lax.broadcasted_iota(jnp.int32, (G, NB), 0)
        oh = jnp.where(gi == bb, 1.0, 0.0)
        su = lax.dot_general(x2, oh, (((1,), (1,)), ((), ())),
                             preferred_element_type=_f32)
        cu = _rows8(jnp.sum(oh, axis=1))
        _acc(su_ref, su, i)
        _acc(cu_ref, cu, i)

    return pl.pallas_call(
        body,
        grid=(NNB,),
        in_specs=[
            pl.BlockSpec((DH, NB), lambda i: (0, i)),
            pl.BlockSpec((DH, DH), lambda i: (0, 0)),
            pl.BlockSpec((DH, 1), lambda i: (0, 0)),
            pl.BlockSpec((8, NB), lambda i: (0, i)),
        ],
        out_specs=[
            pl.BlockSpec((DH, G), lambda i: (0, 0)),
            pl.BlockSpec((8, G), lambda i: (0, 0)),
        ],
        out_shape=[
            jax.ShapeDtypeStruct((DH, G), _f32),
            jax.ShapeDtypeStruct((8, G), _f32),
        ],
    )(h2T, W, cc, batchi)


def _tc_global(suT, cu, gp_cols, Wp, bp2):
    (g0c, b0c, W1, b1c, g1c, be1c, W2, b2c, g2c, be2c, W3, b3c) = gp_cols

    def bnT(h, g, b):
        m = jnp.mean(h, axis=1, keepdims=True)
        v = jnp.mean((h - m) ** 2, axis=1, keepdims=True)
        return g * (h - m) * lax.rsqrt(v + EPS) + b

    def body(su_ref, cu_ref, g0r, b0r, w1r, b1r, g1r, e1r, w2r, b2r, g2r, e2r,
             w3r, b3r, wpr, bpr, out_ref):
        cnt = jnp.maximum(cu_ref[0:1, :], 1.0)
        h = su_ref[...] / cnt
        h = bnT(h, g0r[...], b0r[...])
        h = _lrelu(_dgT(w1r[...], h) + b1r[...])
        h = bnT(h, g1r[...], e1r[...])
        h = _lrelu(_dgT(w2r[...], h) + b2r[...])
        h = bnT(h, g2r[...], e2r[...])
        h = _dgT(w3r[...], h) + b3r[...]
        z = lax.dot_general(h, wpr[...], (((0,), (0,)), ((), ())),
                            preferred_element_type=_f32)
        z = z + bpr[...]
        z = z - jnp.max(z, axis=1, keepdims=True)
        ez = jnp.exp(z)
        out_ref[...] = ez / jnp.sum(ez, axis=1, keepdims=True)

    return pl.pallas_call(
        body,
        out_shape=jax.ShapeDtypeStruct((G, OUTDIM), _f32),
    )(suT, cu, g0c, b0c, W1, b1c, g1c, be1c, W2, b2c, g2c, be2c, W3, b3c,
      Wp, bp2)


def _fold(g, b, mean, var):
    a = g * lax.rsqrt(var + EPS)
    return a, b - a * mean


def _blkdiag(W):
    z = jnp.zeros((DH, DH), _f32)
    return jnp.concatenate(
        [jnp.concatenate([W, z], axis=1), jnp.concatenate([z, W], axis=1)],
        axis=0)


def _st2(st):
    return st[0, :DH] + st[0, DH:], st[1, :DH] + st[1, DH:]


def kernel(x, edge_index, edge_attr, batch, edge_params, node1_params,
           node2_params, global_params, Wp, bp):
    row = edge_index[0]
    col = edge_index[1]
    fE = jnp.float32(E)
    fN = jnp.float32(N)

    xT = jnp.pad(x, ((0, NP - N), (0, 16 - NODE_IN))).T
    attrT = edge_attr.T
    attrTp = (attrT.reshape(EDGE_IN, NEB, BE2, 2)
              .transpose(0, 1, 3, 2).reshape(EDGE_IN, E))
    col_pi = col.reshape(NEB, BE2, 2).transpose(0, 2, 1).reshape(E)
    col_pi4 = col_pi.reshape(NEB, 4, BE4).transpose(0, 2, 1).reshape(E)

    ones_ch = jnp.ones((CH,), _f32)
    zeros_zch = jnp.zeros((ZCH,), _f32)
    zeros_hz32 = jnp.zeros((HZ4, 32), _f32)

    dr2, dc2 = _sc_deg(row, col, ones_ch, zeros_zch)
    degs4 = jnp.stack([dr2[:NP], dc2[:NP], dr2[NP:], dc2[NP:]])

    nm = _tc_node_moments(xT, degs4)
    am = _tc_attr_moments(attrT)
    s_rx, s_rx2 = nm[0, :NODE_IN], nm[1, :NODE_IN]
    s_cx, s_cx2 = nm[2, :NODE_IN], nm[3, :NODE_IN]
    s_a, s_a2 = am[:, 0], am[:, 1]

    eg0, eb0, eW1, eb1, eg1, ebe1, eW2, eb2, eg2, ebe2, eW3, eb3 = edge_params
    m0 = jnp.concatenate([s_rx, s_cx, s_a]) / fE
    q0 = jnp.concatenate([s_rx2, s_cx2, s_a2]) / fE
    a0, c0 = _fold(eg0, eb0, m0, q0 - m0 * m0)
    W1f = a0[:, None] * eW1
    c1 = c0 @ eW1 + eb1
    Ws = jnp.zeros((16, DH), _f32).at[:NODE_IN].set(W1f[:NODE_IN])
    Wd = jnp.zeros((16, DH), _f32).at[:NODE_IN].set(W1f[NODE_IN:2 * NODE_IN])
    Wa = W1f[2 * NODE_IN:]

    PsT, PdT = _tc_proj(xT, Ws, Wd)
    zs, zd = _sc_gath(PsT.T, PdT.T, row, col)
    zs_pk = zs.reshape(E2, 128)
    zd_pk = zd.reshape(E2, 128)

    h1eT, st1 = _tc_edge1(zs_pk, zd_pk, attrTp, Wa, c1[:, None])
    m1 = st1[0] / fE
    a1, c1b = _fold(eg1, ebe1, m1, st1[1] / fE - m1 * m1)
    W2f = a1[:, None] * eW2
    c2 = c1b @ eW2 + eb2

    h2eT, st2, gram = _tc_edge2(h1eT, W2f, c2[:, None])
    m2 = st2[0] / fE
    a2, c2b = _fold(eg2, ebe2, m2, st2[1] / fE - m2 * m2)
    W3f = a2[:, None] * eW3
    c3 = c2b @ eW3 + eb3
    mean_e = m2 @ W3f + c3
    Ee2 = jnp.sum(W3f * (gram @ W3f), axis=0) / fE + 2 * c3 * (m2 @ W3f) + c3 ** 2
    var_e = Ee2 - mean_e ** 2

    ng0, nb0, nW1, nb1, ng1, nbe1, nW2, nb2, ng2, nbe2, nW3, nb3 = node1_params
    m0n = jnp.concatenate([s_rx / fE, mean_e])
    v0n = jnp.concatenate([s_rx2 / fE - (s_rx / fE) ** 2, var_e])
    a0n, c0n = _fold(ng0, nb0, m0n, v0n)
    A9 = a0n[:NODE_IN, None] * nW1[:NODE_IN]
    W9 = W1f[:NODE_IN]
    Mn = W9.T @ jnp.linalg.solve(W9 @ W9.T, A9)
    nW1e = a0n[NODE_IN:, None] * nW1[NODE_IN:]
    B = W3f @ nW1e
    cc = c0n @ nW1 + nb1 + c3 @ nW1e
    h1nT, st1n = _tc_node1(zs_pk, h2eT, Mn, B, cc[:, None])
    m1n = st1n[0] / fE
    a1n, c1n = _fold(ng1, nbe1, m1n, st1n[1] / fE - m1n * m1n)
    nW2f = a1n[:, None] * nW2
    nc2 = c1n @ nW2 + nb2

    h2n_lo, h2n_hi, st2n = _tc_node2(h1nT, nW2f, nc2[:, None])
    m2n = st2n[0] / fE
    a2n, c2n = _fold(ng2, nbe2, m2n, st2n[1] / fE - m2n * m2n)
    nW3f = a2n[:, None] * nW3
    nc3 = c2n @ nW3 + nb3

    S2 = _sc_scatter(col_pi4, h2n_lo.reshape(E, 32), h2n_hi.reshape(E, 32),
                     zeros_hz32)
    SloT = S2[0].T
    ShiT = S2[1].T

    aggT, ast, xst = _tc_agg(SloT, ShiT, degs4, xT,
                             nW3f[:32], nW3f[32:], nc3[:, None])
    mg0, mb0, mW1, mb1, mg1, mbe1, mW2, mb2, mg2, mbe2, mW3, mb3 = node2_params
    mx = xst[0, :NODE_IN] / fN
    vx = xst[1, :NODE_IN] / fN - mx * mx
    ma = ast[0] / fN
    va = ast[1] / fN - ma * ma
    a0m, c0m = _fold(mg0, mb0, jnp.concatenate([mx, ma]),
                     jnp.concatenate([vx, va]))
    Ax = jnp.zeros((16, DH), _f32).at[:NODE_IN].set(
        a0m[:NODE_IN, None] * mW1[:NODE_IN])
    Aagg = a0m[NODE_IN:, None] * mW1[NODE_IN:]
    ccm = (c0m @ mW1 + mb1)[:, None]
    h1mT, st1m = _tc_nmlp_a(xT, aggT, Ax, Aagg, ccm)
    m1m = st1m[0] / fN
    a1m, c1m = _fold(mg1, mbe1, m1m, st1m[1] / fN - m1m * m1m)
    h2mT, st2m = _tc_nmlp_b(h1mT, a1m[:, None] * mW2, (c1m @ mW2 + mb2)[:, None])
    m2m = st2m[0] / fN
    a2m, c2m = _fold(mg2, mbe2, m2m, st2m[1] / fN - m2m * m2m)

    batchi = jnp.broadcast_to(
        jnp.pad(batch, (0, NP - N), constant_values=-1)[None], (8, NP))
    suT, cu = _tc_pool(h2mT, a2m[:, None] * mW3, (c2m @ mW3 + mb3)[:, None],
                       batchi)

    gg0, gb0, gW1, gb1, gg1, gbe1, gW2, gb2, gg2, gbe2, gW3, gb3 = global_params
    gp_cols = (gg0[:, None], gb0[:, None], gW1, gb1[:, None], gg1[:, None],
               gbe1[:, None], gW2, gb2[:, None], gg2[:, None], gbe2[:, None],
               gW3, gb3[:, None])
    return _tc_global(suT, cu, gp_cols, Wp, bp[None])


# speedup vs baseline: 2.3897x; 1.0049x over previous
import functools

import jax
import jax.numpy as jnp
from jax import lax
from jax.experimental import pallas as pl
from jax.experimental.pallas import tpu as pltpu
import jax.experimental.pallas.tpu_sc as plsc

N = 50000
E = 1600000
NODE_IN = 9
EDGE_IN = 12
DH = 64
G = 128
OUTDIM = 6
EPS = 1e-5
SLOPE = 0.1

NP = 50176
E2 = E // 2
E4 = E // 4
BE = 6400
NEB = E // BE
BE2 = BE // 2
BE4 = BE // 4
NB = 6272
NNB = NP // NB
CH = 2000
CHZ = 1000
CHS = 800
EPW = E // 32
EPW2 = E // 16
ZCH = NP // 16
HZ = ZCH // 2
HZ4 = ZCH // 4

_f32 = jnp.float32
_bf16 = jnp.bfloat16


def _mesh():
    return plsc.VectorSubcoreMesh(core_axis_name="c", subcore_axis_name="s")


_SC_PARAMS = pltpu.CompilerParams(use_tc_tiling_on_sc=False)


def _sc_deg(row, col, ones_h_in, zeros_h_in):
    @functools.partial(
        pl.kernel,
        out_type=(jax.ShapeDtypeStruct((2 * NP,), _f32),
                  jax.ShapeDtypeStruct((2 * NP,), _f32)),
        mesh=_mesh(),
        compiler_params=_SC_PARAMS,
        scratch_types=[
            pltpu.VMEM((CH,), jnp.int32),
            pltpu.VMEM((CH,), _f32),
            pltpu.VMEM((ZCH,), _f32),
            pltpu.VMEM_SHARED((NP,), _f32),
            pltpu.VMEM_SHARED((NP,), _f32),
        ],
    )
    def k(row_h, col_h, ones_h, zer_h, outr_h, outc_h,
          idx_v, ones_v, zer_v, acc_r, acc_c):
        cid = lax.axis_index("c")
        sid = lax.axis_index("s")
        wid = sid * 2 + cid
        pltpu.sync_copy(zer_h, zer_v)
        pltpu.sync_copy(zer_v, acc_r.at[pl.ds(sid * ZCH, ZCH)])
        pltpu.sync_copy(zer_v, acc_c.at[pl.ds(sid * ZCH, ZCH)])
        pltpu.sync_copy(ones_h, ones_v)
        plsc.subcore_barrier()

        def step(i, carry):
            base = wid * EPW + i * CH
            pltpu.sync_copy(row_h.at[pl.ds(base, CH)], idx_v)
            pltpu.sync_copy(ones_v, acc_r.at[idx_v], add=True)
            pltpu.sync_copy(col_h.at[pl.ds(base, CH)], idx_v)
            pltpu.sync_copy(ones_v, acc_c.at[idx_v], add=True)
            return carry

        lax.fori_loop(0, EPW // CH, step, 0)
        plsc.subcore_barrier()
        pltpu.sync_copy(acc_r.at[pl.ds(sid * ZCH, ZCH)], zer_v)
        pltpu.sync_copy(zer_v, outr_h.at[pl.ds(cid * NP + sid * ZCH, ZCH)])
        pltpu.sync_copy(acc_c.at[pl.ds(sid * ZCH, ZCH)], zer_v)
        pltpu.sync_copy(zer_v, outc_h.at[pl.ds(cid * NP + sid * ZCH, ZCH)])

    return k(row, col, ones_h_in, zeros_h_in)


def _sc_gath(Ps, Pd, row, col):
    @functools.partial(
        pl.kernel,
        out_type=(jax.ShapeDtypeStruct((E, DH), _bf16),
                  jax.ShapeDtypeStruct((E, DH), _bf16)),
        mesh=_mesh(),
        compiler_params=_SC_PARAMS,
        scratch_types=[
            pltpu.VMEM((CHZ,), jnp.int32),
            pltpu.VMEM((CHZ,), jnp.int32),
            pltpu.VMEM((CHZ, DH), _bf16),
            pltpu.VMEM((CHZ, DH), _bf16),
            pltpu.SemaphoreType.DMA,
            pltpu.SemaphoreType.DMA,
        ],
    )
    def k(ps_h, pd_h, row_h, col_h, zs_h, zd_h,
          idxr_v, idxc_v, rs_v, rd_v, semr, semc):
        cid = lax.axis_index("c")
        sid = lax.axis_index("s")
        wid = sid * 2 + cid

        def step(i, carry):
            base = wid * EPW + i * CHZ
            pltpu.sync_copy(row_h.at[pl.ds(base, CHZ)], idxr_v)
            cr = pltpu.async_copy(ps_h.at[idxr_v], rs_v, semr)
            pltpu.sync_copy(col_h.at[pl.ds(base, CHZ)], idxc_v)
            cc = pltpu.async_copy(pd_h.at[idxc_v], rd_v, semc)
            cr.wait()
            pltpu.sync_copy(rs_v, zs_h.at[pl.ds(base, CHZ)])
            cc.wait()
            pltpu.sync_copy(rd_v, zd_h.at[pl.ds(base, CHZ)])
            return carry

        lax.fori_loop(0, EPW // CHZ, step, 0)

    return k(Ps, Pd, row, col)


def _sc_scatter(col, h_lo, h_hi, zeros_h_in):
    @functools.partial(
        pl.kernel,
        out_type=jax.ShapeDtypeStruct((2, NP, 32), _f32),
        mesh=_mesh(),
        compiler_params=_SC_PARAMS,
        scratch_types=[
            pltpu.VMEM((CHS,), jnp.int32),
            pltpu.VMEM((CHS, 32), _f32),
            pltpu.VMEM_SHARED((NP, 32), _f32),
        ],
    )
    def k(col_h, lo_h, hi_h, zer_h, out_h, idx_v, upd_v, acc):
        cid = lax.axis_index("c")
        sid = lax.axis_index("s")
        pltpu.sync_copy(zer_h, upd_v.at[pl.ds(0, HZ4)])
        for kk in range(4):
            pltpu.sync_copy(upd_v.at[pl.ds(0, HZ4)],
                            acc.at[pl.ds(sid * ZCH + kk * HZ4, HZ4)])
        plsc.subcore_barrier()

        def step_from(h_ref):
            def step(i, carry):
                base = sid * EPW2 + i * CHS
                pltpu.sync_copy(col_h.at[pl.ds(base, CHS)], idx_v)
                pltpu.sync_copy(h_ref.at[pl.ds(base, CHS)], upd_v)
                pltpu.sync_copy(upd_v, acc.at[idx_v], add=True)
                return carry
            return step

        @pl.when(cid == 0)
        def _():
            lax.fori_loop(0, EPW2 // CHS, step_from(lo_h), 0)

        @pl.when(cid == 1)
        def _():
            lax.fori_loop(0, EPW2 // CHS, step_from(hi_h), 0)

        plsc.subcore_barrier()
        for kk in range(4):
            pltpu.sync_copy(acc.at[pl.ds(sid * ZCH + kk * HZ4, HZ4)],
                            upd_v.at[pl.ds(0, HZ4)])
            pltpu.sync_copy(upd_v.at[pl.ds(0, HZ4)],
                            out_h.at[cid, pl.ds(sid * ZCH + kk * HZ4, HZ4)])

    return k(col, h_lo, h_hi, zeros_h_in)


def _dgT(w, hT):
    return lax.dot_general(w, hT, (((0,), (0,)), ((), ())),
                           preferred_element_type=_f32)


def _lrelu(z):
    return jnp.where(z > 0, z, SLOPE * z)


def _rows8(*rows):
    w = rows[0].shape[0]
    pad = jnp.zeros((8 - len(rows), w), _f32)
    return jnp.concatenate([r[None] for r in rows] + [pad], axis=0)


def _acc(ref, blk, i):
    @pl.when(i == 0)
    def _():
        ref[...] = blk

    @pl.when(i > 0)
    def _():
        ref[...] += blk


def _tc_attr_moments(attrT):
    def body(a_ref, st_ref):
        i = pl.program_id(0)
        a = a_ref[...]
        blk = jnp.concatenate(
            [jnp.sum(a, axis=1)[:, None], jnp.sum(a * a, axis=1)[:, None],
             jnp.zeros((EDGE_IN, 14), _f32)], axis=1)
        _acc(st_ref, blk, i)

    return pl.pallas_call(
        body,
        grid=(NEB,),
        in_specs=[pl.BlockSpec((EDGE_IN, BE), lambda i: (0, i))],
        out_specs=pl.BlockSpec((EDGE_IN, 16), lambda i: (0, 0)),
        out_shape=jax.ShapeDtypeStruct((EDGE_IN, 16), _f32),
    )(attrT)


def _tc_node_moments(xT, degs):
    def body(x_ref, d_ref, out_ref):
        xv = x_ref[...]
        deg_r = d_ref[0:1, :] + d_ref[2:3, :]
        deg_c = d_ref[1:2, :] + d_ref[3:4, :]
        out_ref[...] = _rows8(jnp.sum(xv * deg_r, axis=1),
                              jnp.sum(xv * xv * deg_r, axis=1),
                              jnp.sum(xv * deg_c, axis=1),
                              jnp.sum(xv * xv * deg_c, axis=1))

    return pl.pallas_call(
        body,
        out_shape=jax.ShapeDtypeStruct((8, 16), _f32),
    )(xT, degs)


def _tc_proj(xT, Ws, Wd):
    def body(x_ref, ws_ref, wd_ref, ps_ref, pd_ref):
        ps_ref[...] = _dgT(ws_ref[...], x_ref[...]).astype(_bf16)
        pd_ref[...] = _dgT(wd_ref[...], x_ref[...]).astype(_bf16)

    return pl.pallas_call(
        body,
        grid=(NNB,),
        in_specs=[
            pl.BlockSpec((16, NB), lambda i: (0, i)),
            pl.BlockSpec((16, DH), lambda i: (0, 0)),
            pl.BlockSpec((16, DH), lambda i: (0, 0)),
        ],
        out_specs=[
            pl.BlockSpec((DH, NB), lambda i: (0, i)),
            pl.BlockSpec((DH, NB), lambda i: (0, i)),
        ],
        out_shape=[
            jax.ShapeDtypeStruct((DH, NP), _bf16),
            jax.ShapeDtypeStruct((DH, NP), _bf16),
        ],
    )(xT, Ws, Wd)


def _unpack_T(blk):
    lt = jnp.swapaxes(blk[:, :DH], 0, 1)
    rt = jnp.swapaxes(blk[:, DH:], 0, 1)
    return jnp.concatenate([lt, rt], axis=1)


def _stats_T(h):
    return _rows8(jnp.sum(h, axis=1), jnp.sum(h * h, axis=1))


def _tc_edge1(zs_pk, zd_pk, attrTp, Wa, ccol):
    def body(zs_ref, zd_ref, a_ref, wa_ref, c_ref, h_ref, st_ref):
        i = pl.program_id(0)
        zT = _unpack_T(zs_ref[...].astype(_f32) + zd_ref[...].astype(_f32))
        h = _lrelu(zT + _dgT(wa_ref[...], a_ref[...]) + c_ref[...])
        h_ref[...] = h.astype(_bf16)
        _acc(st_ref, _stats_T(h), i)

    return pl.pallas_call(
        body,
        grid=(NEB,),
        in_specs=[
            pl.BlockSpec((BE2, 128), lambda i: (i, 0)),
            pl.BlockSpec((BE2, 128), lambda i: (i, 0)),
            pl.BlockSpec((EDGE_IN, BE), lambda i: (0, i)),
            pl.BlockSpec((EDGE_IN, DH), lambda i: (0, 0)),
            pl.BlockSpec((DH, 1), lambda i: (0, 0)),
        ],
        out_specs=[
            pl.BlockSpec((DH, BE), lambda i: (0, i)),
            pl.BlockSpec((8, DH), lambda i: (0, 0)),
        ],
        out_shape=[
            jax.ShapeDtypeStruct((DH, E), _bf16),
            jax.ShapeDtypeStruct((8, DH), _f32),
        ],
    )(zs_pk, zd_pk, attrTp, Wa, ccol)


def _tc_edge2(h1T, W, ccol):
    def body(h1_ref, w_ref, c_ref, h_ref, st_ref, g_ref):
        i = pl.program_id(0)
        h = _lrelu(_dgT(w_ref[...], h1_ref[...].astype(_f32)) + c_ref[...])
        h_ref[...] = h.astype(_bf16)
        _acc(st_ref, _stats_T(h), i)
        gram = lax.dot_general(h, h, (((1,), (1,)), ((), ())),
                               preferred_element_type=_f32)
        _acc(g_ref, gram, i)

    return pl.pallas_call(
        body,
        grid=(NEB,),
        in_specs=[
            pl.BlockSpec((DH, BE), lambda i: (0, i)),
            pl.BlockSpec((DH, DH), lambda i: (0, 0)),
            pl.BlockSpec((DH, 1), lambda i: (0, 0)),
        ],
        out_specs=[
            pl.BlockSpec((DH, BE), lambda i: (0, i)),
            pl.BlockSpec((8, DH), lambda i: (0, 0)),
            pl.BlockSpec((DH, DH), lambda i: (0, 0)),
        ],
        out_shape=[
            jax.ShapeDtypeStruct((DH, E), _bf16),
            jax.ShapeDtypeStruct((8, DH), _f32),
            jax.ShapeDtypeStruct((DH, DH), _f32),
        ],
    )(h1T, W, ccol)


def _tc_node1(zs_pk, h2T, Mn, B, ccol):
    def body(zs_ref, h2_ref, m_ref, b_ref, c_ref, h_ref, st_ref):
        i = pl.program_id(0)
        zsT = _unpack_T(zs_ref[...].astype(_f32))
        h = _lrelu(_dgT(m_ref[...], zsT)
                   + _dgT(b_ref[...], h2_ref[...].astype(_f32)) + c_ref[...])
        h_ref[...] = h.astype(_bf16)
        _acc(st_ref, _stats_T(h), i)

    return pl.pallas_call(
        body,
        grid=(NEB,),
        in_specs=[
            pl.BlockSpec((BE2, 128), lambda i: (i, 0)),
            pl.BlockSpec((DH, BE), lambda i: (0, i)),
            pl.BlockSpec((DH, DH), lambda i: (0, 0)),
            pl.BlockSpec((DH, DH), lambda i: (0, 0)),
            pl.BlockSpec((DH, 1), lambda i: (0, 0)),
        ],
        out_specs=[
            pl.BlockSpec((DH, BE), lambda i: (0, i)),
            pl.BlockSpec((8, DH), lambda i: (0, 0)),
        ],
        out_shape=[
            jax.ShapeDtypeStruct((DH, E), _bf16),
            jax.ShapeDtypeStruct((8, DH), _f32),
        ],
    )(zs_pk, h2T, Mn, B, ccol)


def _tc_node2(h1T, W, ccol):
    def body(h1_ref, w_ref, c_ref, lo_ref, hi_ref, st_ref):
        i = pl.program_id(0)
        h = _lrelu(_dgT(w_ref[...], h1_ref[...].astype(_f32)) + c_ref[...])
        _acc(st_ref, _stats_T(h), i)
        ht = jnp.swapaxes(h, 0, 1)
        lo = ht[:, :32]
        hi = ht[:, 32:]
        lo_ref[...] = jnp.concatenate(
            [lo[k * BE4:(k + 1) * BE4] for k in range(4)], axis=1)
        hi_ref[...] = jnp.concatenate(
            [hi[k * BE4:(k + 1) * BE4] for k in range(4)], axis=1)

    return pl.pallas_call(
        body,
        grid=(NEB,),
        in_specs=[
            pl.BlockSpec((DH, BE), lambda i: (0, i)),
            pl.BlockSpec((DH, DH), lambda i: (0, 0)),
            pl.BlockSpec((DH, 1), lambda i: (0, 0)),
        ],
        out_specs=[
            pl.BlockSpec((BE4, 128), lambda i: (i, 0)),
            pl.BlockSpec((BE4, 128), lambda i: (i, 0)),
            pl.BlockSpec((8, DH), lambda i: (0, 0)),
        ],
        out_shape=[
            jax.ShapeDtypeStruct((E4, 128), _f32),
            jax.ShapeDtypeStruct((E4, 128), _f32),
            jax.ShapeDtypeStruct((8, DH), _f32),
        ],
    )(h1T, W, ccol)


def _tc_agg(SloT, ShiT, degs, xT, Wlo, Whi, cc):
    def body(lo_ref, hi_ref, d_ref, x_ref, wl_ref, wh_ref, c_ref,
             agg_ref, st_ref, xst_ref):
        i = pl.program_id(0)
        cnt = d_ref[1:2, :] + d_ref[3:4, :]
        sm = _dgT(wl_ref[...], lo_ref[...]) + _dgT(wh_ref[...], hi_ref[...])
        sm = sm + c_ref[...] * cnt
        agg = sm / jnp.maximum(cnt, 1.0)
        agg_ref[...] = agg
        _acc(st_ref, _rows8(jnp.sum(agg, axis=1), jnp.sum(agg * agg, axis=1)), i)
        xv = x_ref[...]
        _acc(xst_ref, _rows8(jnp.sum(xv, axis=1), jnp.sum(xv * xv, axis=1)), i)

    return pl.pallas_call(
        body,
        grid=(NNB,),
        in_specs=[
            pl.BlockSpec((32, NB), lambda i: (0, i)),
            pl.BlockSpec((32, NB), lambda i: (0, i)),
            pl.BlockSpec((4, NB), lambda i: (0, i)),
            pl.BlockSpec((16, NB), lambda i: (0, i)),
            pl.BlockSpec((32, DH), lambda i: (0, 0)),
            pl.BlockSpec((32, DH), lambda i: (0, 0)),
            pl.BlockSpec((DH, 1), lambda i: (0, 0)),
        ],
        out_specs=[
            pl.BlockSpec((DH, NB), lambda i: (0, i)),
            pl.BlockSpec((8, DH), lambda i: (0, 0)),
            pl.BlockSpec((8, 16), lambda i: (0, 0)),
        ],
        out_shape=[
            jax.ShapeDtypeStruct((DH, NP), _f32),
            jax.ShapeDtypeStruct((8, DH), _f32),
            jax.ShapeDtypeStruct((8, 16), _f32),
        ],
    )(SloT, ShiT, degs, xT, Wlo, Whi, cc)


def _tc_nmlp_a(xT, aggT, Ax, Aagg, cc):
    def body(x_ref, agg_ref, ax_ref, aa_ref, c_ref, h_ref, st_ref):
        i = pl.program_id(0)
        h = _lrelu(_dgT(ax_ref[...], x_ref[...]) + _dgT(aa_ref[...], agg_ref[...])
                   + c_ref[...])
        h_ref[...] = h
        pos = lax.broadcasted_iota(jnp.int32, (DH, NB), 1) + i * NB
        hm = jnp.where(pos < N, h, 0.0)
        _acc(st_ref, _rows8(jnp.sum(hm, axis=1), jnp.sum(hm * hm, axis=1)), i)

    return pl.pallas_call(
        body,
        grid=(NNB,),
        in_specs=[
            pl.BlockSpec((16, NB), lambda i: (0, i)),
            pl.BlockSpec((DH, NB), lambda i: (0, i)),
            pl.BlockSpec((16, DH), lambda i: (0, 0)),
            pl.BlockSpec((DH, DH), lambda i: (0, 0)),
            pl.BlockSpec((DH, 1), lambda i: (0, 0)),
        ],
        out_specs=[
            pl.BlockSpec((DH, NB), lambda i: (0, i)),
            pl.BlockSpec((8, DH), lambda i: (0, 0)),
        ],
        out_shape=[
            jax.ShapeDtypeStruct((DH, NP), _f32),
            jax.ShapeDtypeStruct((8, DH), _f32),
        ],
    )(xT, aggT, Ax, Aagg, cc)


def _tc_nmlp_b(hT, W, cc):
    def body(h_ref, w_ref, c_ref, o_ref, st_ref):
        i = pl.program_id(0)
        h = _lrelu(_dgT(w_ref[...], h_ref[...]) + c_ref[...])
        o_ref[...] = h
        pos = lax.broadcasted_iota(jnp.int32, (DH, NB), 1) + i * NB
        hm = jnp.where(pos < N, h, 0.0)
        _acc(st_ref, _rows8(jnp.sum(hm, axis=1), jnp.sum(hm * hm, axis=1)), i)

    return pl.pallas_call(
        body,
        grid=(NNB,),
        in_specs=[
            pl.BlockSpec((DH, NB), lambda i: (0, i)),
            pl.BlockSpec((DH, DH), lambda i: (0, 0)),
            pl.BlockSpec((DH, 1), lambda i: (0, 0)),
        ],
        out_specs=[
            pl.BlockSpec((DH, NB), lambda i: (0, i)),
            pl.BlockSpec((8, DH), lambda i: (0, 0)),
        ],
        out_shape=[
            jax.ShapeDtypeStruct((DH, NP), _f32),
            jax.ShapeDtypeStruct((8, DH), _f32),
        ],
    )(hT, W, cc)


def _tc_pool(h2T, W, cc, batchi):
    def body(h_ref, w_ref, c_ref, b_ref, su_ref, cu_ref):
        i = pl.program_id(0)
        x2 = _dgT(w_ref[...], h_ref[...]) + c_ref[...]
        bb = b_ref[0:1, :]
        gi = lax.broadcasted_iota(jnp.int32, (G, NB), 0)
        oh = jnp.where(gi == bb, 1.0, 0.0)
        su = lax.dot_general(x2, oh, (((1,), (1,)), ((), ())),
                             preferred_element_type=_f32)
        cu = _rows8(jnp.sum(oh, axis=1))
        _acc(su_ref, su, i)
        _acc(cu_ref, cu, i)

    return pl.pallas_call(
        body,
        grid=(NNB,),
        in_specs=[
            pl.BlockSpec((DH, NB), lambda i: (0, i)),
            pl.BlockSpec((DH, DH), lambda i: (0, 0)),
            pl.BlockSpec((DH, 1), lambda i: (0, 0)),
            pl.BlockSpec((8, NB), lambda i: (0, i)),
        ],
        out_specs=[
            pl.BlockSpec((DH, G), lambda i: (0, 0)),
            pl.BlockSpec((8, G), lambda i: (0, 0)),
        ],
        out_shape=[
            jax.ShapeDtypeStruct((DH, G), _f32),
            jax.ShapeDtypeStruct((8, G), _f32),
        ],
    )(h2T, W, cc, batchi)


def _tc_global(suT, cu, gp_cols, Wp, bp2):
    (g0c, b0c, W1, b1c, g1c, be1c, W2, b2c, g2c, be2c, W3, b3c) = gp_cols

    def bnT(h, g, b):
        m = jnp.mean(h, axis=1, keepdims=True)
        v = jnp.mean((h - m) ** 2, axis=1, keepdims=True)
        return g * (h - m) * lax.rsqrt(v + EPS) + b

    def body(su_ref, cu_ref, g0r, b0r, w1r, b1r, g1r, e1r, w2r, b2r, g2r, e2r,
             w3r, b3r, wpr, bpr, out_ref):
        cnt = jnp.maximum(cu_ref[0:1, :], 1.0)
        h = su_ref[...] / cnt
        h = bnT(h, g0r[...], b0r[...])
        h = _lrelu(_dgT(w1r[...], h) + b1r[...])
        h = bnT(h, g1r[...], e1r[...])
        h = _lrelu(_dgT(w2r[...], h) + b2r[...])
        h = bnT(h, g2r[...], e2r[...])
        h = _dgT(w3r[...], h) + b3r[...]
        z = lax.dot_general(h, wpr[...], (((0,), (0,)), ((), ())),
                            preferred_element_type=_f32)
        z = z + bpr[...]
        z = z - jnp.max(z, axis=1, keepdims=True)
        ez = jnp.exp(z)
        out_ref[...] = ez / jnp.sum(ez, axis=1, keepdims=True)

    return pl.pallas_call(
        body,
        out_shape=jax.ShapeDtypeStruct((G, OUTDIM), _f32),
    )(suT, cu, g0c, b0c, W1, b1c, g1c, be1c, W2, b2c, g2c, be2c, W3, b3c,
      Wp, bp2)


def _fold(g, b, mean, var):
    a = g * lax.rsqrt(var + EPS)
    return a, b - a * mean


def kernel(x, edge_index, edge_attr, batch, edge_params, node1_params,
           node2_params, global_params, Wp, bp):
    row = edge_index[0]
    col = edge_index[1]
    fE = jnp.float32(E)
    fN = jnp.float32(N)

    xT = jnp.pad(x, ((0, NP - N), (0, 16 - NODE_IN))).T
    attrT = edge_attr.T
    def _piinv(v):
        return v.reshape(NEB, 2, BE2).transpose(0, 2, 1).reshape(E)

    row_g = _piinv(row)
    col_g = _piinv(col)
    col_pi4 = col.reshape(NEB, 4, BE4).transpose(0, 2, 1).reshape(E)

    ones_ch = jnp.ones((CH,), _f32)
    zeros_zch = jnp.zeros((ZCH,), _f32)
    zeros_hz32 = jnp.zeros((HZ4, 32), _f32)

    dr2, dc2 = _sc_deg(row, col, ones_ch, zeros_zch)
    degs4 = jnp.stack([dr2[:NP], dc2[:NP], dr2[NP:], dc2[NP:]])

    nm = _tc_node_moments(xT, degs4)
    am = _tc_attr_moments(attrT)
    s_rx, s_rx2 = nm[0, :NODE_IN], nm[1, :NODE_IN]
    s_cx, s_cx2 = nm[2, :NODE_IN], nm[3, :NODE_IN]
    s_a, s_a2 = am[:, 0], am[:, 1]

    eg0, eb0, eW1, eb1, eg1, ebe1, eW2, eb2, eg2, ebe2, eW3, eb3 = edge_params
    m0 = jnp.concatenate([s_rx, s_cx, s_a]) / fE
    q0 = jnp.concatenate([s_rx2, s_cx2, s_a2]) / fE
    a0, c0 = _fold(eg0, eb0, m0, q0 - m0 * m0)
    W1f = a0[:, None] * eW1
    c1 = c0 @ eW1 + eb1
    Ws = jnp.zeros((16, DH), _f32).at[:NODE_IN].set(W1f[:NODE_IN])
    Wd = jnp.zeros((16, DH), _f32).at[:NODE_IN].set(W1f[NODE_IN:2 * NODE_IN])
    Wa = W1f[2 * NODE_IN:]

    PsT, PdT = _tc_proj(xT, Ws, Wd)
    zs, zd = _sc_gath(PsT.T, PdT.T, row_g, col_g)
    zs_pk = zs.reshape(E2, 128)
    zd_pk = zd.reshape(E2, 128)

    h1eT, st1 = _tc_edge1(zs_pk, zd_pk, attrT, Wa, c1[:, None])
    m1 = st1[0] / fE
    a1, c1b = _fold(eg1, ebe1, m1, st1[1] / fE - m1 * m1)
    W2f = a1[:, None] * eW2
    c2 = c1b @ eW2 + eb2

    h2eT, st2, gram = _tc_edge2(h1eT, W2f, c2[:, None])
    m2 = st2[0] / fE
    a2, c2b = _fold(eg2, ebe2, m2, st2[1] / fE - m2 * m2)
    W3f = a2[:, None] * eW3
    c3 = c2b @ eW3 + eb3
    mean_e = m2 @ W3f + c3
    Ee2 = jnp.sum(W3f * (gram @ W3f), axis=0) / fE + 2 * c3 * (m2 @ W3f) + c3 ** 2
    var_e = Ee2 - mean_e ** 2

    ng0, nb0, nW1, nb1, ng1, nbe1, nW2, nb2, ng2, nbe2, nW3, nb3 = node1_params
    m0n = jnp.concatenate([s_rx / fE, mean_e])
    v0n = jnp.concatenate([s_rx2 / fE - (s_rx / fE) ** 2, var_e])
    a0n, c0n = _fold(ng0, nb0, m0n, v0n)
    A9 = a0n[:NODE_IN, None] * nW1[:NODE_IN]
    W9 = W1f[:NODE_IN]
    Mn = W9.T @ jnp.linalg.solve(W9 @ W9.T, A9)
    nW1e = a0n[NODE_IN:, None] * nW1[NODE_IN:]
    B = W3f @ nW1e
    cc = c0n @ nW1 + nb1 + c3 @ nW1e
    h1nT, st1n = _tc_node1(zs_pk, h2eT, Mn, B, cc[:, None])
    m1n = st1n[0] / fE
    a1n, c1n = _fold(ng1, nbe1, m1n, st1n[1] / fE - m1n * m1n)
    nW2f = a1n[:, None] * nW2
    nc2 = c1n @ nW2 + nb2

    h2n_lo, h2n_hi, st2n = _tc_node2(h1nT, nW2f, nc2[:, None])
    m2n = st2n[0] / fE
    a2n, c2n = _fold(ng2, nbe2, m2n, st2n[1] / fE - m2n * m2n)
    nW3f = a2n[:, None] * nW3
    nc3 = c2n @ nW3 + nb3

    S2 = _sc_scatter(col_pi4, h2n_lo.reshape(E, 32), h2n_hi.reshape(E, 32),
                     zeros_hz32)
    SloT = S2[0].T
    ShiT = S2[1].T

    aggT, ast, xst = _tc_agg(SloT, ShiT, degs4, xT,
                             nW3f[:32], nW3f[32:], nc3[:, None])
    mg0, mb0, mW1, mb1, mg1, mbe1, mW2, mb2, mg2, mbe2, mW3, mb3 = node2_params
    mx = xst[0, :NODE_IN] / fN
    vx = xst[1, :NODE_IN] / fN - mx * mx
    ma = ast[0] / fN
    va = ast[1] / fN - ma * ma
    a0m, c0m = _fold(mg0, mb0, jnp.concatenate([mx, ma]),
                     jnp.concatenate([vx, va]))
    Ax = jnp.zeros((16, DH), _f32).at[:NODE_IN].set(
        a0m[:NODE_IN, None] * mW1[:NODE_IN])
    Aagg = a0m[NODE_IN:, None] * mW1[NODE_IN:]
    ccm = (c0m @ mW1 + mb1)[:, None]
    h1mT, st1m = _tc_nmlp_a(xT, aggT, Ax, Aagg, ccm)
    m1m = st1m[0] / fN
    a1m, c1m = _fold(mg1, mbe1, m1m, st1m[1] / fN - m1m * m1m)
    h2mT, st2m = _tc_nmlp_b(h1mT, a1m[:, None] * mW2, (c1m @ mW2 + mb2)[:, None])
    m2m = st2m[0] / fN
    a2m, c2m = _fold(mg2, mbe2, m2m, st2m[1] / fN - m2m * m2m)

    batchi = jnp.broadcast_to(
        jnp.pad(batch, (0, NP - N), constant_values=-1)[None], (8, NP))
    suT, cu = _tc_pool(h2mT, a2m[:, None] * mW3, (c2m @ mW3 + mb3)[:, None],
                       batchi)

    gg0, gb0, gW1, gb1, gg1, gbe1, gW2, gb2, gg2, gbe2, gW3, gb3 = global_params
    gp_cols = (gg0[:, None], gb0[:, None], gW1, gb1[:, None], gg1[:, None],
               gbe1[:, None], gW2, gb2[:, None], gg2[:, None], gbe2[:, None],
               gW3, gb3[:, None])
    return _tc_global(suT, cu, gp_cols, Wp, bp[None])


# speedup vs baseline: 2.9356x; 1.2284x over previous
import functools

import jax
import jax.numpy as jnp
from jax import lax
from jax.experimental import pallas as pl
from jax.experimental.pallas import tpu as pltpu
import jax.experimental.pallas.tpu_sc as plsc

N = 50000
E = 1600000
NODE_IN = 9
EDGE_IN = 12
DH = 64
G = 128
OUTDIM = 6
EPS = 1e-5
SLOPE = 0.1

NP = 50176
E2 = E // 2
E4 = E // 4
BE = 6400
NEB = E // BE
BE2 = BE // 2
BE4 = BE // 4
NB = 6272
NNB = NP // NB
CH = 2000
CHZ = 1000
CHS = 400
EPW = E // 32
EPW2 = E // 16
ZCH = NP // 16
HZ = ZCH // 2
HZ8 = ZCH // 8

_f32 = jnp.float32
_bf16 = jnp.bfloat16


def _mesh():
    return plsc.VectorSubcoreMesh(core_axis_name="c", subcore_axis_name="s")


_SC_PARAMS = pltpu.CompilerParams(use_tc_tiling_on_sc=False)


def _sc_deg(row, col, ones_h_in, zeros_h_in):
    @functools.partial(
        pl.kernel,
        out_type=(jax.ShapeDtypeStruct((2 * NP,), _f32),
                  jax.ShapeDtypeStruct((2 * NP,), _f32)),
        mesh=_mesh(),
        compiler_params=_SC_PARAMS,
        scratch_types=[
            pltpu.VMEM((CH,), jnp.int32),
            pltpu.VMEM((CH,), _f32),
            pltpu.VMEM((ZCH,), _f32),
            pltpu.VMEM_SHARED((NP,), _f32),
            pltpu.VMEM_SHARED((NP,), _f32),
        ],
    )
    def k(row_h, col_h, ones_h, zer_h, outr_h, outc_h,
          idx_v, ones_v, zer_v, acc_r, acc_c):
        cid = lax.axis_index("c")
        sid = lax.axis_index("s")
        wid = sid * 2 + cid
        pltpu.sync_copy(zer_h, zer_v)
        pltpu.sync_copy(zer_v, acc_r.at[pl.ds(sid * ZCH, ZCH)])
        pltpu.sync_copy(zer_v, acc_c.at[pl.ds(sid * ZCH, ZCH)])
        pltpu.sync_copy(ones_h, ones_v)
        plsc.subcore_barrier()

        def step(i, carry):
            base = wid * EPW + i * CH
            pltpu.sync_copy(row_h.at[pl.ds(base, CH)], idx_v)
            pltpu.sync_copy(ones_v, acc_r.at[idx_v], add=True)
            pltpu.sync_copy(col_h.at[pl.ds(base, CH)], idx_v)
            pltpu.sync_copy(ones_v, acc_c.at[idx_v], add=True)
            return carry

        lax.fori_loop(0, EPW // CH, step, 0)
        plsc.subcore_barrier()
        pltpu.sync_copy(acc_r.at[pl.ds(sid * ZCH, ZCH)], zer_v)
        pltpu.sync_copy(zer_v, outr_h.at[pl.ds(cid * NP + sid * ZCH, ZCH)])
        pltpu.sync_copy(acc_c.at[pl.ds(sid * ZCH, ZCH)], zer_v)
        pltpu.sync_copy(zer_v, outc_h.at[pl.ds(cid * NP + sid * ZCH, ZCH)])

    return k(row, col, ones_h_in, zeros_h_in)


def _sc_gath(Ps, Pd, row, col):
    @functools.partial(
        pl.kernel,
        out_type=(jax.ShapeDtypeStruct((E, DH), _f32),
                  jax.ShapeDtypeStruct((E, DH), _f32)),
        mesh=_mesh(),
        compiler_params=_SC_PARAMS,
        scratch_types=[
            pltpu.VMEM((CHZ,), jnp.int32),
            pltpu.VMEM((CHZ,), jnp.int32),
            pltpu.VMEM((CHZ, DH), _f32),
            pltpu.VMEM((CHZ, DH), _f32),
            pltpu.SemaphoreType.DMA,
            pltpu.SemaphoreType.DMA,
        ],
    )
    def k(ps_h, pd_h, row_h, col_h, zs_h, zd_h,
          idxr_v, idxc_v, rs_v, rd_v, semr, semc):
        cid = lax.axis_index("c")
        sid = lax.axis_index("s")
        wid = sid * 2 + cid

        def step(i, carry):
            base = wid * EPW + i * CHZ
            pltpu.sync_copy(row_h.at[pl.ds(base, CHZ)], idxr_v)
            cr = pltpu.async_copy(ps_h.at[idxr_v], rs_v, semr)
            pltpu.sync_copy(col_h.at[pl.ds(base, CHZ)], idxc_v)
            cc = pltpu.async_copy(pd_h.at[idxc_v], rd_v, semc)
            cr.wait()
            pltpu.sync_copy(rs_v, zs_h.at[pl.ds(base, CHZ)])
            cc.wait()
            pltpu.sync_copy(rd_v, zd_h.at[pl.ds(base, CHZ)])
            return carry

        lax.fori_loop(0, EPW // CHZ, step, 0)

    return k(Ps, Pd, row, col)


def _sc_scatter(col, h_lo, h_hi, zeros_h_in):
    @functools.partial(
        pl.kernel,
        out_type=jax.ShapeDtypeStruct((2, NP, 32), _f32),
        mesh=_mesh(),
        compiler_params=_SC_PARAMS,
        scratch_types=[
            pltpu.VMEM((CHS,), jnp.int32),
            pltpu.VMEM((CHS,), jnp.int32),
            pltpu.VMEM((CHS, 32), _f32),
            pltpu.VMEM((CHS, 32), _f32),
            pltpu.SemaphoreType.DMA,
            pltpu.SemaphoreType.DMA,
            pltpu.VMEM_SHARED((NP, 32), _f32),
        ],
    )
    def k(col_h, lo_h, hi_h, zer_h, out_h, idx0_v, idx1_v, upd0_v, upd1_v,
          sem0, sem1, acc):
        cid = lax.axis_index("c")
        sid = lax.axis_index("s")

        if True:
            pltpu.sync_copy(zer_h, upd0_v.at[pl.ds(0, HZ8)])
            for kk in range(8):
                pltpu.sync_copy(upd0_v.at[pl.ds(0, HZ8)],
                                acc.at[pl.ds(sid * ZCH + kk * HZ8, HZ8)])
            plsc.subcore_barrier()

            def loop_over(h_ref):
                def load(i, idx_v, upd_v):
                    base = sid * EPW2 + i * CHS
                    pltpu.sync_copy(col_h.at[pl.ds(base, CHS)], idx_v)
                    pltpu.sync_copy(h_ref.at[pl.ds(base, CHS)], upd_v)

                nst = EPW2 // CHS
                load(0, idx0_v, upd0_v)

                def step(i, carry):
                    even = lax.rem(i, 2) == 0

                    @pl.when(even)
                    def _():
                        d = pltpu.async_copy(upd0_v, acc.at[idx0_v], sem0,
                                             add=True)
                        @pl.when(i + 1 < nst)
                        def _():
                            load(i + 1, idx1_v, upd1_v)
                        d.wait()

                    @pl.when(jnp.logical_not(even))
                    def _():
                        d = pltpu.async_copy(upd1_v, acc.at[idx1_v], sem1,
                                             add=True)
                        @pl.when(i + 1 < nst)
                        def _():
                            load(i + 1, idx0_v, upd0_v)
                        d.wait()

                    return carry

                lax.fori_loop(0, nst, step, 0)

            @pl.when(cid == 0)
            def _():
                loop_over(lo_h)

            @pl.when(cid == 1)
            def _():
                loop_over(hi_h)

            plsc.subcore_barrier()
            for kk in range(8):
                pltpu.sync_copy(acc.at[pl.ds(sid * ZCH + kk * HZ8, HZ8)],
                                upd0_v.at[pl.ds(0, HZ8)])
                pltpu.sync_copy(upd0_v.at[pl.ds(0, HZ8)],
                                out_h.at[cid, pl.ds(sid * ZCH + kk * HZ8, HZ8)])

    return k(col, h_lo, h_hi, zeros_h_in)


def _dgT(w, hT):
    return lax.dot_general(w, hT, (((0,), (0,)), ((), ())),
                           preferred_element_type=_f32)


def _lrelu(z):
    return jnp.where(z > 0, z, SLOPE * z)


def _rows8(*rows):
    w = rows[0].shape[0]
    pad = jnp.zeros((8 - len(rows), w), _f32)
    return jnp.concatenate([r[None] for r in rows] + [pad], axis=0)


def _acc(ref, blk, i):
    @pl.when(i == 0)
    def _():
        ref[...] = blk

    @pl.when(i > 0)
    def _():
        ref[...] += blk


def _tc_attr_moments(attrT):
    def body(a_ref, st_ref):
        i = pl.program_id(0)
        a = a_ref[...]
        blk = jnp.concatenate(
            [jnp.sum(a, axis=1)[:, None], jnp.sum(a * a, axis=1)[:, None],
             jnp.zeros((EDGE_IN, 14), _f32)], axis=1)
        _acc(st_ref, blk, i)

    return pl.pallas_call(
        body,
        grid=(NEB,),
        in_specs=[pl.BlockSpec((EDGE_IN, BE), lambda i: (0, i))],
        out_specs=pl.BlockSpec((EDGE_IN, 16), lambda i: (0, 0)),
        out_shape=jax.ShapeDtypeStruct((EDGE_IN, 16), _f32),
    )(attrT)


def _tc_node_moments(xT, degs):
    def body(x_ref, d_ref, out_ref):
        xv = x_ref[...]
        deg_r = d_ref[0:1, :] + d_ref[2:3, :]
        deg_c = d_ref[1:2, :] + d_ref[3:4, :]
        out_ref[...] = _rows8(jnp.sum(xv * deg_r, axis=1),
                              jnp.sum(xv * xv * deg_r, axis=1),
                              jnp.sum(xv * deg_c, axis=1),
                              jnp.sum(xv * xv * deg_c, axis=1))

    return pl.pallas_call(
        body,
        out_shape=jax.ShapeDtypeStruct((8, 16), _f32),
    )(xT, degs)


def _tc_proj(xT, Ws, Wd):
    def body(x_ref, ws_ref, wd_ref, ps_ref, pd_ref):
        ps_ref[...] = _dgT(ws_ref[...], x_ref[...])
        pd_ref[...] = _dgT(wd_ref[...], x_ref[...])

    return pl.pallas_call(
        body,
        grid=(NNB,),
        in_specs=[
            pl.BlockSpec((16, NB), lambda i: (0, i)),
            pl.BlockSpec((16, DH), lambda i: (0, 0)),
            pl.BlockSpec((16, DH), lambda i: (0, 0)),
        ],
        out_specs=[
            pl.BlockSpec((DH, NB), lambda i: (0, i)),
            pl.BlockSpec((DH, NB), lambda i: (0, i)),
        ],
        out_shape=[
            jax.ShapeDtypeStruct((DH, NP), _f32),
            jax.ShapeDtypeStruct((DH, NP), _f32),
        ],
    )(xT, Ws, Wd)


def _stats_T(h):
    return _rows8(jnp.sum(h, axis=1), jnp.sum(h * h, axis=1))


def _unpack_T(blk):
    lt = jnp.swapaxes(blk[:, :DH], 0, 1)
    rt = jnp.swapaxes(blk[:, DH:], 0, 1)
    return jnp.concatenate([lt, rt], axis=1)


def _tc_edge1(zs_pk, zd_pk, attrTp, Wa, ccol):
    def body(zs_ref, zd_ref, a_ref, wa_ref, c_ref, h_ref, st_ref):
        i = pl.program_id(0)
        zT = _unpack_T(zs_ref[...] + zd_ref[...])
        h = _lrelu(zT + _dgT(wa_ref[...], a_ref[...]) + c_ref[...])
        h_ref[...] = h.astype(_bf16)
        _acc(st_ref, _stats_T(h), i)

    return pl.pallas_call(
        body,
        grid=(NEB,),
        in_specs=[
            pl.BlockSpec((BE2, 128), lambda i: (i, 0)),
            pl.BlockSpec((BE2, 128), lambda i: (i, 0)),
            pl.BlockSpec((EDGE_IN, BE), lambda i: (0, i)),
            pl.BlockSpec((EDGE_IN, DH), lambda i: (0, 0)),
            pl.BlockSpec((DH, 1), lambda i: (0, 0)),
        ],
        out_specs=[
            pl.BlockSpec((DH, BE), lambda i: (0, i)),
            pl.BlockSpec((8, DH), lambda i: (0, 0)),
        ],
        out_shape=[
            jax.ShapeDtypeStruct((DH, E), _bf16),
            jax.ShapeDtypeStruct((8, DH), _f32),
        ],
    )(zs_pk, zd_pk, attrTp, Wa, ccol)


def _tc_edge2(h1T, W, ccol):
    def body(h1_ref, w_ref, c_ref, h_ref, st_ref, g_ref):
        i = pl.program_id(0)
        h = _lrelu(_dgT(w_ref[...], h1_ref[...].astype(_f32)) + c_ref[...])
        h_ref[...] = h.astype(_bf16)
        _acc(st_ref, _stats_T(h), i)
        gram = lax.dot_general(h, h, (((1,), (1,)), ((), ())),
                               preferred_element_type=_f32)
        _acc(g_ref, gram, i)

    return pl.pallas_call(
        body,
        grid=(NEB,),
        in_specs=[
            pl.BlockSpec((DH, BE), lambda i: (0, i)),
            pl.BlockSpec((DH, DH), lambda i: (0, 0)),
            pl.BlockSpec((DH, 1), lambda i: (0, 0)),
        ],
        out_specs=[
            pl.BlockSpec((DH, BE), lambda i: (0, i)),
            pl.BlockSpec((8, DH), lambda i: (0, 0)),
            pl.BlockSpec((DH, DH), lambda i: (0, 0)),
        ],
        out_shape=[
            jax.ShapeDtypeStruct((DH, E), _bf16),
            jax.ShapeDtypeStruct((8, DH), _f32),
            jax.ShapeDtypeStruct((DH, DH), _f32),
        ],
    )(h1T, W, ccol)


def _tc_node1(zs_pk, h2T, Mn, B, ccol):
    def body(zs_ref, h2_ref, m_ref, b_ref, c_ref, h_ref, st_ref):
        i = pl.program_id(0)
        zsT = _unpack_T(zs_ref[...])
        h = _lrelu(_dgT(m_ref[...], zsT)
                   + _dgT(b_ref[...], h2_ref[...].astype(_f32)) + c_ref[...])
        h_ref[...] = h.astype(_bf16)
        _acc(st_ref, _stats_T(h), i)

    return pl.pallas_call(
        body,
        grid=(NEB,),
        in_specs=[
            pl.BlockSpec((BE2, 128), lambda i: (i, 0)),
            pl.BlockSpec((DH, BE), lambda i: (0, i)),
            pl.BlockSpec((DH, DH), lambda i: (0, 0)),
            pl.BlockSpec((DH, DH), lambda i: (0, 0)),
            pl.BlockSpec((DH, 1), lambda i: (0, 0)),
        ],
        out_specs=[
            pl.BlockSpec((DH, BE), lambda i: (0, i)),
            pl.BlockSpec((8, DH), lambda i: (0, 0)),
        ],
        out_shape=[
            jax.ShapeDtypeStruct((DH, E), _bf16),
            jax.ShapeDtypeStruct((8, DH), _f32),
        ],
    )(zs_pk, h2T, Mn, B, ccol)


def _tc_node2(h1T, W, ccol):
    def body(h1_ref, w_ref, c_ref, lo_ref, hi_ref, st_ref):
        i = pl.program_id(0)
        h = _lrelu(_dgT(w_ref[...], h1_ref[...].astype(_f32)) + c_ref[...])
        _acc(st_ref, _stats_T(h), i)
        ht = jnp.swapaxes(h, 0, 1)
        lo = ht[:, :32]
        hi = ht[:, 32:]
        lo_ref[...] = jnp.concatenate(
            [lo[k * BE4:(k + 1) * BE4] for k in range(4)], axis=1)
        hi_ref[...] = jnp.concatenate(
            [hi[k * BE4:(k + 1) * BE4] for k in range(4)], axis=1)

    return pl.pallas_call(
        body,
        grid=(NEB,),
        in_specs=[
            pl.BlockSpec((DH, BE), lambda i: (0, i)),
            pl.BlockSpec((DH, DH), lambda i: (0, 0)),
            pl.BlockSpec((DH, 1), lambda i: (0, 0)),
        ],
        out_specs=[
            pl.BlockSpec((BE4, 128), lambda i: (i, 0)),
            pl.BlockSpec((BE4, 128), lambda i: (i, 0)),
            pl.BlockSpec((8, DH), lambda i: (0, 0)),
        ],
        out_shape=[
            jax.ShapeDtypeStruct((E4, 128), _f32),
            jax.ShapeDtypeStruct((E4, 128), _f32),
            jax.ShapeDtypeStruct((8, DH), _f32),
        ],
    )(h1T, W, ccol)


def _tc_agg(SloT, ShiT, degs, xT, Wlo, Whi, cc):
    def body(lo_ref, hi_ref, d_ref, x_ref, wl_ref, wh_ref, c_ref,
             agg_ref, st_ref, xst_ref):
        i = pl.program_id(0)
        cnt = d_ref[1:2, :] + d_ref[3:4, :]
        sm = _dgT(wl_ref[...], lo_ref[...]) + _dgT(wh_ref[...], hi_ref[...])
        sm = sm + c_ref[...] * cnt
        agg = sm / jnp.maximum(cnt, 1.0)
        agg_ref[...] = agg
        _acc(st_ref, _rows8(jnp.sum(agg, axis=1), jnp.sum(agg * agg, axis=1)), i)
        xv = x_ref[...]
        _acc(xst_ref, _rows8(jnp.sum(xv, axis=1), jnp.sum(xv * xv, axis=1)), i)

    return pl.pallas_call(
        body,
        grid=(NNB,),
        in_specs=[
            pl.BlockSpec((32, NB), lambda i: (0, i)),
            pl.BlockSpec((32, NB), lambda i: (0, i)),
            pl.BlockSpec((4, NB), lambda i: (0, i)),
            pl.BlockSpec((16, NB), lambda i: (0, i)),
            pl.BlockSpec((32, DH), lambda i: (0, 0)),
            pl.BlockSpec((32, DH), lambda i: (0, 0)),
            pl.BlockSpec((DH, 1), lambda i: (0, 0)),
        ],
        out_specs=[
            pl.BlockSpec((DH, NB), lambda i: (0, i)),
            pl.BlockSpec((8, DH), lambda i: (0, 0)),
            pl.BlockSpec((8, 16), lambda i: (0, 0)),
        ],
        out_shape=[
            jax.ShapeDtypeStruct((DH, NP), _f32),
            jax.ShapeDtypeStruct((8, DH), _f32),
            jax.ShapeDtypeStruct((8, 16), _f32),
        ],
    )(SloT, ShiT, degs, xT, Wlo, Whi, cc)


def _tc_nmlp_a(xT, aggT, Ax, Aagg, cc):
    def body(x_ref, agg_ref, ax_ref, aa_ref, c_ref, h_ref, st_ref):
        i = pl.program_id(0)
        h = _lrelu(_dgT(ax_ref[...], x_ref[...]) + _dgT(aa_ref[...], agg_ref[...])
                   + c_ref[...])
        h_ref[...] = h
        pos = lax.broadcasted_iota(jnp.int32, (DH, NB), 1) + i * NB
        hm = jnp.where(pos < N, h, 0.0)
        _acc(st_ref, _rows8(jnp.sum(hm, axis=1), jnp.sum(hm * hm, axis=1)), i)

    return pl.pallas_call(
        body,
        grid=(NNB,),
        in_specs=[
            pl.BlockSpec((16, NB), lambda i: (0, i)),
            pl.BlockSpec((DH, NB), lambda i: (0, i)),
            pl.BlockSpec((16, DH), lambda i: (0, 0)),
            pl.BlockSpec((DH, DH), lambda i: (0, 0)),
            pl.BlockSpec((DH, 1), lambda i: (0, 0)),
        ],
        out_specs=[
            pl.BlockSpec((DH, NB), lambda i: (0, i)),
            pl.BlockSpec((8, DH), lambda i: (0, 0)),
        ],
        out_shape=[
            jax.ShapeDtypeStruct((DH, NP), _f32),
            jax.ShapeDtypeStruct((8, DH), _f32),
        ],
    )(xT, aggT, Ax, Aagg, cc)


def _tc_nmlp_b(hT, W, cc):
    def body(h_ref, w_ref, c_ref, o_ref, st_ref):
        i = pl.program_id(0)
        h = _lrelu(_dgT(w_ref[...], h_ref[...]) + c_ref[...])
        o_ref[...] = h
        pos = lax.broadcasted_iota(jnp.int32, (DH, NB), 1) + i * NB
        hm = jnp.where(pos < N, h, 0.0)
        _acc(st_ref, _rows8(jnp.sum(hm, axis=1), jnp.sum(hm * hm, axis=1)), i)

    return pl.pallas_call(
        body,
        grid=(NNB,),
        in_specs=[
            pl.BlockSpec((DH, NB), lambda i: (0, i)),
            pl.BlockSpec((DH, DH), lambda i: (0, 0)),
            pl.BlockSpec((DH, 1), lambda i: (0, 0)),
        ],
        out_specs=[
            pl.BlockSpec((DH, NB), lambda i: (0, i)),
            pl.BlockSpec((8, DH), lambda i: (0, 0)),
        ],
        out_shape=[
            jax.ShapeDtypeStruct((DH, NP), _f32),
            jax.ShapeDtypeStruct((8, DH), _f32),
        ],
    )(hT, W, cc)


def _tc_pool(h2T, W, cc, batchi):
    def body(h_ref, w_ref, c_ref, b_ref, su_ref, cu_ref):
        i = pl.program_id(0)
        x2 = _dgT(w_ref[...], h_ref[...]) + c_ref[...]
        bb = b_ref[0:1, :]
        gi = lax.broadcasted_iota(jnp.int32, (G, NB), 0)
        oh = jnp.where(gi == bb, 1.0, 0.0)
        su = lax.dot_general(x2, oh, (((1,), (1,)), ((), ())),
                             preferred_element_type=_f32)
        cu = _rows8(jnp.sum(oh, axis=1))
        _acc(su_ref, su, i)
        _acc(cu_ref, cu, i)

    return pl.pallas_call(
        body,
        grid=(NNB,),
        in_specs=[
            pl.BlockSpec((DH, NB), lambda i: (0, i)),
            pl.BlockSpec((DH, DH), lambda i: (0, 0)),
            pl.BlockSpec((DH, 1), lambda i: (0, 0)),
            pl.BlockSpec((8, NB), lambda i: (0, i)),
        ],
        out_specs=[
            pl.BlockSpec((DH, G), lambda i: (0, 0)),
            pl.BlockSpec((8, G), lambda i: (0, 0)),
        ],
        out_shape=[
            jax.ShapeDtypeStruct((DH, G), _f32),
            jax.ShapeDtypeStruct((8, G), _f32),
        ],
    )(h2T, W, cc, batchi)


def _tc_global(suT, cu, gp_cols, Wp, bp2):
    (g0c, b0c, W1, b1c, g1c, be1c, W2, b2c, g2c, be2c, W3, b3c) = gp_cols

    def bnT(h, g, b):
        m = jnp.mean(h, axis=1, keepdims=True)
        v = jnp.mean((h - m) ** 2, axis=1, keepdims=True)
        return g * (h - m) * lax.rsqrt(v + EPS) + b

    def body(su_ref, cu_ref, g0r, b0r, w1r, b1r, g1r, e1r, w2r, b2r, g2r, e2r,
             w3r, b3r, wpr, bpr, out_ref):
        cnt = jnp.maximum(cu_ref[0:1, :], 1.0)
        h = su_ref[...] / cnt
        h = bnT(h, g0r[...], b0r[...])
        h = _lrelu(_dgT(w1r[...], h) + b1r[...])
        h = bnT(h, g1r[...], e1r[...])
        h = _lrelu(_dgT(w2r[...], h) + b2r[...])
        h = bnT(h, g2r[...], e2r[...])
        h = _dgT(w3r[...], h) + b3r[...]
        z = lax.dot_general(h, wpr[...], (((0,), (0,)), ((), ())),
                            preferred_element_type=_f32)
        z = z + bpr[...]
        z = z - jnp.max(z, axis=1, keepdims=True)
        ez = jnp.exp(z)
        out_ref[...] = ez / jnp.sum(ez, axis=1, keepdims=True)

    return pl.pallas_call(
        body,
        out_shape=jax.ShapeDtypeStruct((G, OUTDIM), _f32),
    )(suT, cu, g0c, b0c, W1, b1c, g1c, be1c, W2, b2c, g2c, be2c, W3, b3c,
      Wp, bp2)


def _fold(g, b, mean, var):
    a = g * lax.rsqrt(var + EPS)
    return a, b - a * mean


def kernel(x, edge_index, edge_attr, batch, edge_params, node1_params,
           node2_params, global_params, Wp, bp):
    row = edge_index[0]
    col = edge_index[1]
    fE = jnp.float32(E)
    fN = jnp.float32(N)

    xT = jnp.pad(x, ((0, NP - N), (0, 16 - NODE_IN))).T
    attrT = edge_attr.T
    def _piinv(v):
        return v.reshape(NEB, 2, BE2).transpose(0, 2, 1).reshape(E)

    row_g = _piinv(row)
    col_g = _piinv(col)
    col_pi4 = col.reshape(NEB, 4, BE4).transpose(0, 2, 1).reshape(E)

    ones_ch = jnp.ones((CH,), _f32)
    zeros_zch = jnp.zeros((ZCH,), _f32)
    zeros_hz32 = jnp.zeros((HZ8, 32), _f32)

    dr2, dc2 = _sc_deg(row, col, ones_ch, zeros_zch)
    degs4 = jnp.stack([dr2[:NP], dc2[:NP], dr2[NP:], dc2[NP:]])

    nm = _tc_node_moments(xT, degs4)
    am = _tc_attr_moments(attrT)
    s_rx, s_rx2 = nm[0, :NODE_IN], nm[1, :NODE_IN]
    s_cx, s_cx2 = nm[2, :NODE_IN], nm[3, :NODE_IN]
    s_a, s_a2 = am[:, 0], am[:, 1]

    eg0, eb0, eW1, eb1, eg1, ebe1, eW2, eb2, eg2, ebe2, eW3, eb3 = edge_params
    m0 = jnp.concatenate([s_rx, s_cx, s_a]) / fE
    q0 = jnp.concatenate([s_rx2, s_cx2, s_a2]) / fE
    a0, c0 = _fold(eg0, eb0, m0, q0 - m0 * m0)
    W1f = a0[:, None] * eW1
    c1 = c0 @ eW1 + eb1
    Ws = jnp.zeros((16, DH), _f32).at[:NODE_IN].set(W1f[:NODE_IN])
    Wd = jnp.zeros((16, DH), _f32).at[:NODE_IN].set(W1f[NODE_IN:2 * NODE_IN])
    Wa = W1f[2 * NODE_IN:]

    PsT, PdT = _tc_proj(xT, Ws, Wd)
    zs, zd = _sc_gath(PsT.T, PdT.T, row_g, col_g)
    zs_pk = zs.reshape(E2, 128)
    zd_pk = zd.reshape(E2, 128)

    h1eT, st1 = _tc_edge1(zs_pk, zd_pk, attrT, Wa, c1[:, None])
    m1 = st1[0] / fE
    a1, c1b = _fold(eg1, ebe1, m1, st1[1] / fE - m1 * m1)
    W2f = a1[:, None] * eW2
    c2 = c1b @ eW2 + eb2

    h2eT, st2, gram = _tc_edge2(h1eT, W2f, c2[:, None])
    m2 = st2[0] / fE
    a2, c2b = _fold(eg2, ebe2, m2, st2[1] / fE - m2 * m2)
    W3f = a2[:, None] * eW3
    c3 = c2b @ eW3 + eb3
    mean_e = m2 @ W3f + c3
    Ee2 = jnp.sum(W3f * (gram @ W3f), axis=0) / fE + 2 * c3 * (m2 @ W3f) + c3 ** 2
    var_e = Ee2 - mean_e ** 2

    ng0, nb0, nW1, nb1, ng1, nbe1, nW2, nb2, ng2, nbe2, nW3, nb3 = node1_params
    m0n = jnp.concatenate([s_rx / fE, mean_e])
    v0n = jnp.concatenate([s_rx2 / fE - (s_rx / fE) ** 2, var_e])
    a0n, c0n = _fold(ng0, nb0, m0n, v0n)
    A9 = a0n[:NODE_IN, None] * nW1[:NODE_IN]
    W9 = W1f[:NODE_IN]
    Mn = W9.T @ jnp.linalg.solve(W9 @ W9.T, A9)
    nW1e = a0n[NODE_IN:, None] * nW1[NODE_IN:]
    B = W3f @ nW1e
    cc = c0n @ nW1 + nb1 + c3 @ nW1e
    h1nT, st1n = _tc_node1(zs_pk, h2eT, Mn, B, cc[:, None])
    m1n = st1n[0] / fE
    a1n, c1n = _fold(ng1, nbe1, m1n, st1n[1] / fE - m1n * m1n)
    nW2f = a1n[:, None] * nW2
    nc2 = c1n @ nW2 + nb2

    h2n_lo, h2n_hi, st2n = _tc_node2(h1nT, nW2f, nc2[:, None])
    m2n = st2n[0] / fE
    a2n, c2n = _fold(ng2, nbe2, m2n, st2n[1] / fE - m2n * m2n)
    nW3f = a2n[:, None] * nW3
    nc3 = c2n @ nW3 + nb3

    S2 = _sc_scatter(col_pi4, h2n_lo.reshape(E, 32), h2n_hi.reshape(E, 32),
                     zeros_hz32)
    SloT = S2[0].T
    ShiT = S2[1].T

    aggT, ast, xst = _tc_agg(SloT, ShiT, degs4, xT,
                             nW3f[:32], nW3f[32:], nc3[:, None])
    mg0, mb0, mW1, mb1, mg1, mbe1, mW2, mb2, mg2, mbe2, mW3, mb3 = node2_params
    mx = xst[0, :NODE_IN] / fN
    vx = xst[1, :NODE_IN] / fN - mx * mx
    ma = ast[0] / fN
    va = ast[1] / fN - ma * ma
    a0m, c0m = _fold(mg0, mb0, jnp.concatenate([mx, ma]),
                     jnp.concatenate([vx, va]))
    Ax = jnp.zeros((16, DH), _f32).at[:NODE_IN].set(
        a0m[:NODE_IN, None] * mW1[:NODE_IN])
    Aagg = a0m[NODE_IN:, None] * mW1[NODE_IN:]
    ccm = (c0m @ mW1 + mb1)[:, None]
    h1mT, st1m = _tc_nmlp_a(xT, aggT, Ax, Aagg, ccm)
    m1m = st1m[0] / fN
    a1m, c1m = _fold(mg1, mbe1, m1m, st1m[1] / fN - m1m * m1m)
    h2mT, st2m = _tc_nmlp_b(h1mT, a1m[:, None] * mW2, (c1m @ mW2 + mb2)[:, None])
    m2m = st2m[0] / fN
    a2m, c2m = _fold(mg2, mbe2, m2m, st2m[1] / fN - m2m * m2m)

    batchi = jnp.broadcast_to(
        jnp.pad(batch, (0, NP - N), constant_values=-1)[None], (8, NP))
    suT, cu = _tc_pool(h2mT, a2m[:, None] * mW3, (c2m @ mW3 + mb3)[:, None],
                       batchi)

    gg0, gb0, gW1, gb1, gg1, gbe1, gW2, gb2, gg2, gbe2, gW3, gb3 = global_params
    gp_cols = (gg0[:, None], gb0[:, None], gW1, gb1[:, None], gg1[:, None],
               gbe1[:, None], gW2, gb2[:, None], gg2[:, None], gbe2[:, None],
               gW3, gb3[:, None])
    return _tc_global(suT, cu, gp_cols, Wp, bp[None])
